# Initial kernel scaffold; baseline (speedup 1.0000x reference)
import jax
import jax.numpy as jnp
from jax.experimental import pallas as pl


def kernel(grid_features, grid_coords, query_coords, N):
    raise NotImplementedError("write your pallas kernel here")


# speedup vs baseline: 1.9912x; 1.9912x over previous
import jax
import jax.numpy as jnp
from jax.experimental import pallas as pl
from jax.experimental.pallas import tpu as pltpu

Q = 512
HW = 256
C = 128
ROWS = 16


def _nn_broadcast_kernel(q_ref, gc_ref, gf_ref, out_ref, feat_ref):
    @pl.when(pl.program_id(0) == 0)
    def _compute():
        q = q_ref[...]
        gc = gc_ref[...]
        qx = q[:, 0:1]
        qy = q[:, 1:2]
        gx = gc[0:1, :]
        gy = gc[1:2, :]
        dx = qx - gx
        dy = qy - gy
        d = jnp.sqrt(dx * dx + dy * dy)
        idx = jnp.argmin(d, axis=1)
        onehot = (idx[:, None] == jax.lax.broadcasted_iota(
            jnp.int32, (Q, HW), 1)).astype(jnp.float32)
        feat_ref[...] = jnp.dot(onehot, gf_ref[...],
                                preferred_element_type=jnp.float32)
    out_ref[...] = jnp.broadcast_to(feat_ref[...][None], (ROWS, Q, C))


def kernel(grid_features, grid_coords, query_coords, N):
    gf = jnp.transpose(grid_features, (0, 2, 3, 1)).reshape(HW, C)
    gc = grid_coords.reshape(2, HW)
    out = pl.pallas_call(
        _nn_broadcast_kernel,
        grid=(Q // ROWS,),
        in_specs=[
            pl.BlockSpec((Q, 2), lambda i: (0, 0)),
            pl.BlockSpec((2, HW), lambda i: (0, 0)),
            pl.BlockSpec((HW, C), lambda i: (0, 0)),
        ],
        out_specs=pl.BlockSpec((ROWS, Q, C), lambda i: (i, 0, 0)),
        out_shape=jax.ShapeDtypeStruct((Q, Q, C), jnp.float32),
        scratch_shapes=[pltpu.VMEM((Q, C), jnp.float32)],
    )(query_coords, gc, gf)
    return out.reshape(1, Q, 1, Q, 1, C)
